# baseline (speedup 1.0000x reference)
import functools

import jax
import jax.numpy as jnp
from jax import lax
from jax.experimental import pallas as pl
from jax.experimental.pallas import tpu as pltpu
from jax.experimental.pallas import tpu_sc as plsc

DIM = 64
B = 16384 * 50

_info = plsc.get_sparse_core_info()
NC = _info.num_cores
NS = _info.num_subcores
NW = NC * NS
B_PER_W = B // NW
CHUNK = 1024
N_CHUNKS = B_PER_W // CHUNK

_mesh = plsc.VectorSubcoreMesh(core_axis_name="c", subcore_axis_name="s")


@functools.partial(
    pl.kernel,
    mesh=_mesh,
    out_type=jax.ShapeDtypeStruct((B, DIM), jnp.float32),
    scratch_types=[
        pltpu.VMEM((CHUNK,), jnp.int32),
        pltpu.VMEM((CHUNK, DIM), jnp.float32),
        pltpu.SemaphoreType.DMA,
    ],
    compiler_params=pltpu.CompilerParams(use_tc_tiling_on_sc=False),
)
def _gather(idx_hbm, table_hbm, out_hbm, idx_v, rows_v, sem):
    wid = lax.axis_index("s") * NC + lax.axis_index("c")
    base = wid * B_PER_W

    def body(i, _):
        off = base + i * CHUNK
        pltpu.sync_copy(idx_hbm.at[pl.ds(off, CHUNK)], idx_v)
        pltpu.async_copy(table_hbm.at[idx_v], rows_v, sem).wait()
        pltpu.sync_copy(rows_v, out_hbm.at[pl.ds(off, CHUNK)])
        return 0

    lax.fori_loop(0, N_CHUNKS, body, 0)


def kernel(tokens, table):
    flat = tokens.reshape(-1).astype(jnp.int32)
    return _gather(flat, table)


# speedup vs baseline: 1.0188x; 1.0188x over previous
import functools

import jax
import jax.numpy as jnp
from jax import lax
from jax.experimental import pallas as pl
from jax.experimental.pallas import tpu as pltpu
from jax.experimental.pallas import tpu_sc as plsc

DIM = 64
B = 16384 * 50

_info = plsc.get_sparse_core_info()
NC = _info.num_cores
NS = _info.num_subcores
NW = NC * NS
B_PER_W = B // NW
CHUNK = 256
N_CHUNKS = B_PER_W // CHUNK
NBUF = 4
N_OUTER = N_CHUNKS // NBUF

_mesh = plsc.VectorSubcoreMesh(core_axis_name="c", subcore_axis_name="s")


@functools.partial(
    pl.kernel,
    mesh=_mesh,
    out_type=jax.ShapeDtypeStruct((B, DIM), jnp.float32),
    scratch_types=[
        pltpu.VMEM((N_CHUNKS, CHUNK), jnp.int32),
        *[pltpu.VMEM((CHUNK, DIM), jnp.float32) for _ in range(NBUF)],
        *[pltpu.SemaphoreType.DMA for _ in range(2 * NBUF)],
    ],
    compiler_params=pltpu.CompilerParams(use_tc_tiling_on_sc=False),
)
def _gather(idx_hbm, table_hbm, out_hbm, idx_all,
            r0, r1, r2, r3, g0, g1, g2, g3, s0, s1, s2, s3):
    rows = (r0, r1, r2, r3)
    semG = (g0, g1, g2, g3)
    semS = (s0, s1, s2, s3)
    wid = lax.axis_index("s") * NC + lax.axis_index("c")
    base = wid * B_PER_W

    pltpu.sync_copy(idx_hbm.at[wid], idx_all)

    def start_gather(i, b):
        pltpu.async_copy(table_hbm.at[idx_all.at[i]], rows[b], semG[b])

    def wait_gather(i, b):
        pltpu.make_async_copy(table_hbm.at[idx_all.at[i]], rows[b], semG[b]).wait()

    def start_store(i, b):
        pltpu.async_copy(rows[b], out_hbm.at[pl.ds(base + i * CHUNK, CHUNK)], semS[b])

    def wait_store(i, b):
        pltpu.make_async_copy(rows[b], out_hbm.at[pl.ds(base + i * CHUNK, CHUNK)], semS[b]).wait()

    for b in range(NBUF):
        start_gather(b, b)
    wait_gather(0, 0)
    start_store(0, 0)
    wait_gather(1, 1)
    start_store(1, 1)

    @pl.loop(1, N_OUTER)
    def _body(g):
        for b in range(NBUF):
            i = g * NBUF + b
            wait_store(i - NBUF, b)
            start_gather(i, b)
            bj = (b + 2) % NBUF
            wait_gather(i - 2, bj)
            start_store(i - 2, bj)

    n = N_CHUNKS
    wait_gather(n - 2, (n - 2) % NBUF)
    start_store(n - 2, (n - 2) % NBUF)
    wait_gather(n - 1, (n - 1) % NBUF)
    start_store(n - 1, (n - 1) % NBUF)
    for k in range(NBUF):
        i = n - NBUF + k
        wait_store(i, i % NBUF)


def kernel(tokens, table):
    flat = tokens.reshape(NW, N_CHUNKS, CHUNK).astype(jnp.int32)
    return _gather(flat, table)
